# baseline (speedup 1.0000x reference)
import functools

import jax
import jax.numpy as jnp
from jax import lax
from jax.experimental import pallas as pl
from jax.experimental.pallas import tpu as pltpu
from jax.experimental.pallas import tpu_sc as plsc

B = 4096
NC_FEAT = 13
F = 26
V = 100000
D = 64
H1, H2, H3 = 512, 256, 128

FPC = F // 2
NSLAB = FPC * 8
VC = 5760
NCH = 17
VLAST = 2048
VTAIL = 99968
NVREG = B // 16


def _chunk_pass(idx_v, cbuf, out_acc, dsplats, lo, clen, zero):
    hi = lo + clen

    def body(i, _):
        iv = idx_v[pl.ds(i * 16, 16)]
        m = (iv >= lo) & (iv < hi)
        ic = jnp.clip(iv - lo, 0, clen - 1)
        for d in range(8):
            g = plsc.load_gather(cbuf, [dsplats[d], ic], mask=m)
            plsc.addupdate(out_acc.at[d, pl.ds(i * 16, 16)], jnp.where(m, g, zero))
        return 0

    lax.fori_loop(0, NVREG, body, 0)


def _emb_body(tab_hbm, tail_hbm, idx_hbm, out_hbm,
              idx_v, cbuf0, cbuf1, tbuf, out_acc,
              sem_a, sem_b, sem_t):
    c = lax.axis_index("c")
    tid = lax.axis_index("s")
    nslab_t = jnp.where(tid < 8, 7, 6)
    cbufs = (cbuf0, cbuf1)
    csems = (sem_a, sem_b)
    zero = jnp.zeros((16,), jnp.float32)
    dsplats = [jnp.full((16,), d, jnp.int32) for d in range(8)]

    def do_slab(k, _):
        s = tid + 16 * k
        row0 = pl.multiple_of((c * FPC + (s >> 3)) * D + (s & 7) * 8, 8)

        def chunk_src(ci, clen):
            return tab_hbm.at[pl.ds(row0, 8), pl.ds(ci * VC, clen)]

        off = pl.multiple_of((c * FPC + (s >> 3)) * B, 8)
        pltpu.sync_copy(idx_hbm.at[pl.ds(off, B)], idx_v)

        def zbody(i, _):
            out_acc[0, pl.ds(i * 16, 16)] = zero
            return 0

        def zrow(d, _):
            def zb(i, _):
                out_acc[d, pl.ds(i * 16, 16)] = zero
                return 0
            lax.fori_loop(0, NVREG, zb, 0)
            return 0

        lax.fori_loop(0, 8, zrow, 0)

        pltpu.async_copy(chunk_src(0, VC), cbuf0, sem_a)
        pltpu.async_copy(tail_hbm.at[pl.ds(row0, 8), :], tbuf, sem_t)
        for ci in range(NCH + 1):
            clen = VC if ci < NCH else VLAST
            buf = ci & 1
            if ci < NCH:
                nlen = VC if ci + 1 < NCH else VLAST
                pltpu.async_copy(
                    chunk_src(ci + 1, nlen),
                    cbufs[buf ^ 1].at[:, pl.ds(0, nlen)],
                    csems[buf ^ 1],
                )
            pltpu.make_async_copy(
                chunk_src(ci, clen), cbufs[buf].at[:, pl.ds(0, clen)],
                csems[buf]
            ).wait()
            _chunk_pass(idx_v, cbufs[buf], out_acc, dsplats,
                        ci * VC, clen, zero)
        pltpu.make_async_copy(
            tail_hbm.at[pl.ds(row0, 8), :], tbuf, sem_t
        ).wait()
        _chunk_pass(idx_v, tbuf, out_acc, dsplats, VTAIL, 32, zero)
        pltpu.sync_copy(out_acc, out_hbm.at[c * NSLAB + s])
        return 0

    lax.fori_loop(0, nslab_t, do_slab, 0)


def _sc_embed_t(tab2d, tail128, idx_flat):
    mesh = plsc.VectorSubcoreMesh(core_axis_name="c", subcore_axis_name="s")
    k = functools.partial(
        pl.kernel,
        mesh=mesh,
        out_type=jax.ShapeDtypeStruct((2 * NSLAB, 8, B), jnp.float32),
        scratch_types=[
            pltpu.VMEM((B,), jnp.int32),
            pltpu.VMEM((8, VC), jnp.float32),
            pltpu.VMEM((8, VC), jnp.float32),
            pltpu.VMEM((8, 128), jnp.float32),
            pltpu.VMEM((8, B), jnp.float32),
            pltpu.SemaphoreType.DMA,
            pltpu.SemaphoreType.DMA,
            pltpu.SemaphoreType.DMA,
        ],
        compiler_params=pltpu.CompilerParams(needs_layout_passes=False),
    )(_emb_body)
    return k(tab2d, tail128, idx_flat)


def _mlp_body(cont_ref, emb_ref, wc_ref, bc_ref, w1a_ref, w1b_ref, b1_ref,
              w2_ref, b2_ref, w3_ref, b3_ref, wo_ref, bo_ref, out_ref):
    f32 = jnp.float32
    dot = lambda a, b: jnp.dot(a, b, preferred_element_type=f32)
    xt = dot(wc_ref[...], cont_ref[...]) + bc_ref[...]
    h = dot(w1a_ref[...], xt) + dot(w1b_ref[...], emb_ref[...]) + b1_ref[...]
    h = jnp.maximum(h, 0.0)
    h = jnp.maximum(dot(w2_ref[...], h) + b2_ref[...], 0.0)
    h = jnp.maximum(dot(w3_ref[...], h) + b3_ref[...], 0.0)
    logit = dot(wo_ref[...], h) + bo_ref[...]
    out_ref[...] = jax.nn.sigmoid(logit)


def _tc_mlp_t(cont_t, emb_t, WcT, bcC, W1aT, W1bT, b1C, W2T, b2C, W3T, b3C,
              WoT, boC):
    BB = 512
    grid = (B // BB,)
    full = lambda a: pl.BlockSpec(a.shape, lambda i: (0,) * a.ndim)
    return pl.pallas_call(
        _mlp_body,
        grid=grid,
        in_specs=[
            pl.BlockSpec((NC_FEAT, BB), lambda i: (0, i)),
            pl.BlockSpec((F * D, BB), lambda i: (0, i)),
            full(WcT), full(bcC), full(W1aT), full(W1bT), full(b1C),
            full(W2T), full(b2C), full(W3T), full(b3C), full(WoT), full(boC),
        ],
        out_specs=pl.BlockSpec((1, BB), lambda i: (0, i)),
        out_shape=jax.ShapeDtypeStruct((1, B), jnp.float32),
    )(cont_t, emb_t, WcT, bcC, W1aT, W1bT, b1C, W2T, b2C, W3T, b3C, WoT, boC)


def kernel(continuous_features, categorical_features, W_cont, b_cont, tables,
           W1, b1, W2, b2, W3, b3, Wo, bo):
    tab2d = tables.transpose(0, 2, 1).reshape(F * D, V)
    tail128 = jnp.pad(lax.slice(tab2d, (0, 99968), (F * D, V)),
                      ((0, 0), (0, 96)))
    idx_flat = categorical_features.astype(jnp.int32).T.reshape(F * B)
    emb3 = _sc_embed_t(tab2d, tail128, idx_flat)
    emb_t = emb3.reshape(F * D, B)

    W1T = W1.T
    out_t = _tc_mlp_t(
        continuous_features.T, emb_t,
        W_cont.T, b_cont.reshape(D, 1),
        W1T[:, :D], W1T[:, D:], b1.reshape(H1, 1),
        W2.T, b2.reshape(H2, 1),
        W3.T, b3.reshape(H3, 1),
        Wo.T, bo.reshape(1, 1),
    )
    return out_t.reshape(B, 1)


# speedup vs baseline: 2.3522x; 2.3522x over previous
import functools

import jax
import jax.numpy as jnp
from jax import lax
from jax.experimental import pallas as pl
from jax.experimental.pallas import tpu as pltpu
from jax.experimental.pallas import tpu_sc as plsc

B = 4096
NC_FEAT = 13
F = 26
V = 100000
D = 64
H1, H2, H3 = 512, 256, 128

FPC = F // 2
NSLAB = FPC * 8
L0 = 49920
L1 = 50048
Q = 24960
Q1 = 25088
NVREG = B // 16


def _gather_pass(idx_v, row_v, out_v, lo, hi, mx, one_sided_lo, first):
    zero = jnp.zeros((16,), jnp.float32)

    def body(i, _):
        iv = idx_v[pl.ds(i * 16, 16)]
        if one_sided_lo:
            m = iv >= lo
        elif lo == 0:
            m = iv < hi
        else:
            m = (iv >= lo) & (iv < hi)
        ic = jnp.clip(iv - lo, 0, mx)
        g = plsc.load_gather(row_v, [ic])
        prev = zero if first else out_v[pl.ds(i * 16, 16)]
        out_v[pl.ds(i * 16, 16)] = jnp.where(m, g, prev)
        return 0

    lax.fori_loop(0, NVREG, body, 0)


def _emb_body(tab_hbm, tail_hbm, idx_hbm, out_hbm,
              idx_v, row_v, out_v,
              stage0, stage1, tslab0, tslab1, ostage0, ostage1,
              sem_a, sem_b, sem_ta, sem_tb):
    c = lax.axis_index("c")
    tid = lax.axis_index("s")
    dd = tid & 7
    q = tid >> 3
    stages = (stage0, stage1)
    ssems = (sem_a, sem_b)
    tslabs = (tslab0, tslab1)
    tsems = (sem_ta, sem_tb)
    ostages = (ostage0, ostage1)

    def row0(s):
        return pl.multiple_of((c * FPC + (s >> 3)) * D + (s & 7) * 8, 8)

    def slab_src(s, h):
        if h == 0:
            return tab_hbm.at[pl.ds(row0(s), 8), pl.ds(0, L0)]
        return tab_hbm.at[pl.ds(row0(s), 8), pl.ds(L0, L1)]

    def tail_src(s):
        return tail_hbm.at[pl.ds(row0(s), 8), :]

    @pl.when(tid == 0)
    def _():
        pltpu.async_copy(slab_src(0, 0), stage0, sem_a)
        pltpu.async_copy(tail_src(0), tslab0, sem_ta)

    def step(s, h, par):
        @pl.when(tid == 0)
        def _():
            pltpu.make_async_copy(slab_src(s, h), stages[h], ssems[h]).wait()
            if h == 1:
                pltpu.make_async_copy(tail_src(s), tslabs[par],
                                      tsems[par]).wait()

        plsc.subcore_barrier()

        @pl.when(tid == 0)
        def _():
            if h == 0:
                pltpu.async_copy(slab_src(s, 1), stages[1], ssems[1])

                @pl.when(s > 0)
                def _():
                    pltpu.sync_copy(ostages[par ^ 1],
                                    out_hbm.at[c * NSLAB + s - 1])
            else:
                @pl.when(s + 1 < NSLAB)
                def _():
                    pltpu.async_copy(slab_src(s + 1, 0), stages[0], ssems[0])
                    pltpu.async_copy(tail_src(s + 1), tslabs[par ^ 1],
                                     tsems[par ^ 1])

        if h == 0:
            @pl.when((s & 7) == 0)
            def _():
                off = pl.multiple_of((c * FPC + (s >> 3)) * B, 8)
                pltpu.sync_copy(idx_hbm.at[pl.ds(off, B)], idx_v)

            @pl.when(q == 0)
            def _():
                pltpu.sync_copy(stage0.at[dd, pl.ds(0, Q)],
                                row_v.at[pl.ds(0, Q)])
                _gather_pass(idx_v, row_v, out_v, 0, Q, Q - 1, False, True)

            @pl.when(q == 1)
            def _():
                pltpu.sync_copy(stage0.at[dd, pl.ds(Q, Q)],
                                row_v.at[pl.ds(0, Q)])
                _gather_pass(idx_v, row_v, out_v, Q, L0, Q - 1, False, True)
        else:
            @pl.when(q == 0)
            def _():
                pltpu.sync_copy(stage1.at[dd, pl.ds(0, Q)],
                                row_v.at[pl.ds(0, Q)])
                _gather_pass(idx_v, row_v, out_v, L0, L0 + Q, Q - 1,
                             False, False)

            @pl.when(q == 1)
            def _():
                pltpu.sync_copy(stage1.at[dd, pl.ds(Q, Q1)],
                                row_v.at[pl.ds(0, Q1)])
                pltpu.sync_copy(tslabs[par].at[dd],
                                row_v.at[pl.ds(Q1, 128)])
                _gather_pass(idx_v, row_v, out_v, L0 + Q, V, Q1 + 127,
                             True, False)

            pltpu.sync_copy(out_v, ostages[par].at[dd * 2 + q])

    def outer(p, _):
        step(2 * p, 0, 0)
        step(2 * p, 1, 0)
        step(2 * p + 1, 0, 1)
        step(2 * p + 1, 1, 1)
        return 0

    lax.fori_loop(0, NSLAB // 2, outer, 0)
    plsc.subcore_barrier()

    @pl.when(tid == 0)
    def _():
        pltpu.sync_copy(ostage1, out_hbm.at[c * NSLAB + NSLAB - 1])


def _sc_embed_t(tab2d, tail128, idx_flat):
    mesh = plsc.VectorSubcoreMesh(core_axis_name="c", subcore_axis_name="s")
    k = functools.partial(
        pl.kernel,
        mesh=mesh,
        out_type=jax.ShapeDtypeStruct((2 * NSLAB, 16, B), jnp.float32),
        scratch_types=[
            pltpu.VMEM((B,), jnp.int32),
            pltpu.VMEM((Q1 + 128,), jnp.float32),
            pltpu.VMEM((B,), jnp.float32),
            pltpu.VMEM_SHARED((8, L0), jnp.float32),
            pltpu.VMEM_SHARED((8, L1), jnp.float32),
            pltpu.VMEM_SHARED((8, 128), jnp.float32),
            pltpu.VMEM_SHARED((8, 128), jnp.float32),
            pltpu.VMEM_SHARED((16, B), jnp.float32),
            pltpu.VMEM_SHARED((16, B), jnp.float32),
            pltpu.SemaphoreType.DMA,
            pltpu.SemaphoreType.DMA,
            pltpu.SemaphoreType.DMA,
            pltpu.SemaphoreType.DMA,
        ],
        compiler_params=pltpu.CompilerParams(needs_layout_passes=False),
    )(_emb_body)
    return k(tab2d, tail128, idx_flat)


def _mlp_body(cont_ref, emb_ref, wc_ref, bc_ref, w1a_ref, w1b_ref, b1_ref,
              w2_ref, b2_ref, w3_ref, b3_ref, wo_ref, bo_ref, out_ref):
    f32 = jnp.float32
    dot = lambda a, b: jnp.dot(a, b, preferred_element_type=f32)
    xt = dot(wc_ref[...], cont_ref[...]) + bc_ref[...]
    h = dot(w1a_ref[...], xt) + dot(w1b_ref[...], emb_ref[...]) + b1_ref[...]
    h = jnp.maximum(h, 0.0)
    h = jnp.maximum(dot(w2_ref[...], h) + b2_ref[...], 0.0)
    h = jnp.maximum(dot(w3_ref[...], h) + b3_ref[...], 0.0)
    logit = dot(wo_ref[...], h) + bo_ref[...]
    out_ref[...] = jax.nn.sigmoid(logit)


def _tc_mlp_t(cont_t, emb_t, WcT, bcC, W1aT, W1bT, b1C, W2T, b2C, W3T, b3C,
              WoT, boC):
    BB = 512
    grid = (B // BB,)
    full = lambda a: pl.BlockSpec(a.shape, lambda i: (0,) * a.ndim)
    return pl.pallas_call(
        _mlp_body,
        grid=grid,
        in_specs=[
            pl.BlockSpec((NC_FEAT, BB), lambda i: (0, i)),
            pl.BlockSpec((2 * F * D, BB), lambda i: (0, i)),
            full(WcT), full(bcC), full(W1aT), full(W1bT), full(b1C),
            full(W2T), full(b2C), full(W3T), full(b3C), full(WoT), full(boC),
        ],
        out_specs=pl.BlockSpec((1, BB), lambda i: (0, i)),
        out_shape=jax.ShapeDtypeStruct((1, B), jnp.float32),
    )(cont_t, emb_t, WcT, bcC, W1aT, W1bT, b1C, W2T, b2C, W3T, b3C, WoT, boC)


def kernel(continuous_features, categorical_features, W_cont, b_cont, tables,
           W1, b1, W2, b2, W3, b3, Wo, bo):
    tab2d = tables.transpose(0, 2, 1).reshape(F * D, V)
    tail128 = jnp.pad(lax.slice(tab2d, (0, 99968), (F * D, V)),
                      ((0, 0), (0, 96)))
    idx_flat = categorical_features.astype(jnp.int32).T.reshape(F * B)
    emb3 = _sc_embed_t(tab2d, tail128, idx_flat)
    emb2 = emb3.reshape(2 * F * D, B)

    W1T = W1.T
    W1bT2 = jnp.repeat(W1T[:, D:], 2, axis=1)
    out_t = _tc_mlp_t(
        continuous_features.T, emb2,
        W_cont.T, b_cont.reshape(D, 1),
        W1T[:, :D], W1bT2, b1.reshape(H1, 1),
        W2.T, b2.reshape(H2, 1),
        W3.T, b3.reshape(H3, 1),
        Wo.T, bo.reshape(1, 1),
    )
    return out_t.reshape(B, 1)
